# baseline (speedup 1.0000x reference)
import functools

import jax
import jax.numpy as jnp
from jax import lax
from jax.experimental import pallas as pl
from jax.experimental.pallas import tpu as pltpu
from jax.experimental.pallas import tpu_sc as plsc

N_NODES = 10000
N_EDGES = 320000
D = 128
DH = D // 2
BN_EPS = 1e-5

NC, NS = 2, 16
CHUNK = 128
N_PAD = 10240
ROWS_PT = N_PAD // NS
NBUF = 5
GEDGE = 160
NE_PT = N_EDGES // NS
N_GRP = NE_PT // GEDGE

_sc_mesh = plsc.VectorSubcoreMesh(core_axis_name="c", subcore_axis_name="s")
_sc_params = pltpu.CompilerParams(use_tc_tiling_on_sc=False,
                                  skip_device_barrier=True)


@functools.partial(
    pl.kernel,
    out_type=jax.ShapeDtypeStruct((NC, N_PAD, 16), jnp.float32),
    mesh=_sc_mesh,
    compiler_params=_sc_params,
    scratch_types=[
        pltpu.VMEM((NE_PT,), jnp.int32),
        pltpu.VMEM((GEDGE, 16), jnp.float32),
        pltpu.VMEM((ROWS_PT, 16), jnp.float32),
        pltpu.VMEM_SHARED((N_PAD, 16), jnp.float32),
    ],
)
def _degree_kernel(edge_hbm, out_hbm, idx_v, ones_v, stage_v, deg_sh):
    c = lax.axis_index("c")
    s = lax.axis_index("s")

    def fill(r, _):
        ones_v[r, :] = jnp.ones((16,), jnp.float32)
        return _

    lax.fori_loop(0, GEDGE, fill, 0)

    def zrow(r, _):
        stage_v[r, :] = jnp.zeros((16,), jnp.float32)
        return _

    lax.fori_loop(0, ROWS_PT, zrow, 0)
    pltpu.sync_copy(stage_v, deg_sh.at[pl.ds(s * ROWS_PT, ROWS_PT)])
    plsc.subcore_barrier()

    pltpu.sync_copy(edge_hbm.at[c, s], idx_v)

    def body(j, _):
        pltpu.sync_copy(
            ones_v, deg_sh.at[idx_v.at[pl.ds(j * GEDGE, GEDGE)]], add=True)
        return _

    lax.fori_loop(0, N_GRP, body, 0)
    plsc.subcore_barrier()
    pltpu.sync_copy(deg_sh.at[pl.ds(s * ROWS_PT, ROWS_PT)], stage_v)
    pltpu.sync_copy(stage_v, out_hbm.at[c, pl.ds(s * ROWS_PT, ROWS_PT)])


@functools.partial(
    pl.kernel,
    out_type=jax.ShapeDtypeStruct((NC, N_PAD, DH), jnp.bfloat16),
    mesh=_sc_mesh,
    compiler_params=_sc_params,
    scratch_types=[
        pltpu.VMEM((NE_PT,), jnp.int32),
        pltpu.VMEM((NE_PT,), jnp.int32),
        [pltpu.VMEM((GEDGE, DH), jnp.bfloat16)] * NBUF,
        pltpu.VMEM_SHARED((N_PAD, DH), jnp.bfloat16),
        pltpu.VMEM_SHARED((N_PAD, DH), jnp.bfloat16),
        [pltpu.SemaphoreType.DMA] * NBUF,
    ],
)
def _segsum_kernel(u_hbm, src_hbm, dst_hbm, out_hbm,
                   src_v, dst_v, rows_bufs, agg_sh, u_sh, sems):
    c = lax.axis_index("c")
    s = lax.axis_index("s")

    stage_v = rows_bufs[0]
    _spans = tuple((i * GEDGE, GEDGE) for i in range(ROWS_PT // GEDGE))

    def zrow(r, _):
        for k in range(DH // 32):
            stage_v[r, pl.ds(k * 32, 32)] = jnp.zeros((32,), jnp.bfloat16)
        return _

    lax.fori_loop(0, GEDGE, zrow, 0)
    for off, ln in _spans:
        pltpu.sync_copy(stage_v.at[pl.ds(0, ln)],
                        agg_sh.at[pl.ds(s * ROWS_PT + off, ln)])
    for off, ln in _spans:
        sl = pl.ds(s * ROWS_PT + off, ln)
        pltpu.sync_copy(u_hbm.at[c, sl], rows_bufs[1].at[pl.ds(0, ln)])
        pltpu.sync_copy(rows_bufs[1].at[pl.ds(0, ln)], u_sh.at[sl])
    plsc.subcore_barrier()

    pltpu.sync_copy(src_hbm.at[s], src_v)
    pltpu.sync_copy(dst_hbm.at[s], dst_v)

    def _span(j):
        return pl.ds(j * GEDGE, GEDGE)

    def _table(b):
        return u_sh

    for b in range(NBUF):
        pltpu.async_copy(
            _table(b).at[src_v.at[_span(b)]], rows_bufs[b], sems[b])

    def body(i, _):
        for b in range(NBUF):
            j = i * NBUF + b
            pltpu.make_async_copy(
                _table(b).at[src_v.at[_span(j)]], rows_bufs[b],
                sems[b]).wait()
            pltpu.sync_copy(
                rows_bufs[b], agg_sh.at[dst_v.at[_span(j)]], add=True)

            @pl.when(j + NBUF < N_GRP)
            def _issue():
                pltpu.async_copy(
                    _table(b).at[src_v.at[_span(j + NBUF)]],
                    rows_bufs[b], sems[b])
        return _

    lax.fori_loop(0, N_GRP // NBUF, body, 0)
    plsc.subcore_barrier()
    for off, ln in _spans:
        pltpu.sync_copy(agg_sh.at[pl.ds(s * ROWS_PT + off, ln)],
                        stage_v.at[pl.ds(0, ln)])
        pltpu.sync_copy(stage_v.at[pl.ds(0, ln)],
                        out_hbm.at[c, pl.ds(s * ROWS_PT + off, ln)])


def _norm_col(deg16):
    d = deg16[:N_NODES, 0:1]
    return jnp.where(d > 0, lax.rsqrt(jnp.maximum(d, 1e-12)), 0.0)


def _split_u(u_ref, un):
    ub = un.astype(jnp.bfloat16)
    u_ref[0, :N_NODES, :] = ub[:, :DH]
    u_ref[1, :N_NODES, :] = ub[:, DH:]
    zpad = jnp.zeros((N_PAD - N_NODES, DH), jnp.bfloat16)
    u_ref[0, N_NODES:, :] = zpad
    u_ref[1, N_NODES:, :] = zpad


def _scale_body(deg_ref, x_ref, u_ref):
    _split_u(u_ref, x_ref[...] * _norm_col(deg_ref[0]))


_scale_call = pl.pallas_call(
    _scale_body,
    out_shape=jax.ShapeDtypeStruct((NC, N_PAD, DH), jnp.bfloat16),
)


def _post_body(with_u, aggp_ref, h_ref, deg_ref, wg_ref, bg_ref, wl_ref,
               bl_ref, gm_ref, bt_ref, *outs):
    agg = jnp.concatenate(
        [aggp_ref[0][:N_NODES], aggp_ref[1][:N_NODES]],
        axis=1).astype(jnp.float32)
    h = h_ref[...]
    norm_in = _norm_col(deg_ref[1])
    gcn = jnp.dot(agg, wg_ref[...], preferred_element_type=jnp.float32)
    gcn = gcn * norm_in + bg_ref[...]
    lin = jnp.dot(h, wl_ref[...], preferred_element_type=jnp.float32) + bl_ref[...]
    out = gcn + h + lin
    mean = jnp.mean(out, axis=0, keepdims=True)
    cen = out - mean
    var = jnp.mean(cen * cen, axis=0, keepdims=True)
    hn = cen * lax.rsqrt(var + BN_EPS) * gm_ref[...] + bt_ref[...]
    hn = jnp.maximum(hn, 0.0)
    outs[0][...] = hn
    if with_u:
        _split_u(outs[1], hn * _norm_col(deg_ref[0]))


_post_first = pl.pallas_call(
    functools.partial(_post_body, True),
    out_shape=[jax.ShapeDtypeStruct((N_NODES, D), jnp.float32),
               jax.ShapeDtypeStruct((NC, N_PAD, DH), jnp.bfloat16)],
)
_post_last = pl.pallas_call(
    functools.partial(_post_body, False),
    out_shape=[jax.ShapeDtypeStruct((N_NODES, D), jnp.float32)],
)


def kernel(x, edge_index, W_gcn, b_gcn, W_lin, b_lin, gamma, beta):
    edge_f = edge_index.reshape(2, NS, NE_PT)
    src_r = edge_f[0]
    dst_r = edge_f[1]
    deg16 = _degree_kernel(edge_f)
    u = _scale_call(deg16, x)
    h = x
    for l in range(2):
        aggp = _segsum_kernel(u, src_r, dst_r)
        args = (aggp, h, deg16, W_gcn[l], b_gcn[l].reshape(1, D),
                W_lin[l], b_lin[l].reshape(1, D), gamma[l].reshape(1, D),
                beta[l].reshape(1, D))
        if l == 0:
            h, u = _post_first(*args)
        else:
            (h,) = _post_last(*args)
    return h


# speedup vs baseline: 1.0282x; 1.0282x over previous
import functools

import jax
import jax.numpy as jnp
from jax import lax
from jax.experimental import pallas as pl
from jax.experimental.pallas import tpu as pltpu
from jax.experimental.pallas import tpu_sc as plsc

N_NODES = 10000
N_EDGES = 320000
D = 128
DH = D // 2
BN_EPS = 1e-5

NC, NS = 2, 16
CHUNK = 128
N_PAD = 10240
ROWS_PT = N_PAD // NS
NBUF = 2
GEDGE = 400
NE_PT = N_EDGES // NS
N_GRP = NE_PT // GEDGE

_sc_mesh = plsc.VectorSubcoreMesh(core_axis_name="c", subcore_axis_name="s")
_sc_params = pltpu.CompilerParams(use_tc_tiling_on_sc=False)


@functools.partial(
    pl.kernel,
    out_type=jax.ShapeDtypeStruct((NC, N_PAD, 16), jnp.float32),
    mesh=_sc_mesh,
    compiler_params=_sc_params,
    scratch_types=[
        pltpu.VMEM((NE_PT,), jnp.int32),
        pltpu.VMEM((GEDGE, 16), jnp.float32),
        pltpu.VMEM((ROWS_PT, 16), jnp.float32),
        pltpu.VMEM_SHARED((N_PAD, 16), jnp.float32),
    ],
)
def _degree_kernel(edge_hbm, out_hbm, idx_v, ones_v, stage_v, deg_sh):
    c = lax.axis_index("c")
    s = lax.axis_index("s")

    def fill(r, _):
        ones_v[r, :] = jnp.ones((16,), jnp.float32)
        return _

    lax.fori_loop(0, GEDGE, fill, 0)

    def zrow(r, _):
        stage_v[r, :] = jnp.zeros((16,), jnp.float32)
        return _

    lax.fori_loop(0, ROWS_PT, zrow, 0)
    pltpu.sync_copy(stage_v, deg_sh.at[pl.ds(s * ROWS_PT, ROWS_PT)])
    plsc.subcore_barrier()

    pltpu.sync_copy(edge_hbm.at[c, s], idx_v)

    def body(j, _):
        pltpu.sync_copy(
            ones_v, deg_sh.at[idx_v.at[pl.ds(j * GEDGE, GEDGE)]], add=True)
        return _

    lax.fori_loop(0, N_GRP, body, 0)
    plsc.subcore_barrier()
    pltpu.sync_copy(deg_sh.at[pl.ds(s * ROWS_PT, ROWS_PT)], stage_v)
    pltpu.sync_copy(stage_v, out_hbm.at[c, pl.ds(s * ROWS_PT, ROWS_PT)])


@functools.partial(
    pl.kernel,
    out_type=jax.ShapeDtypeStruct((NC, N_PAD, DH), jnp.bfloat16),
    mesh=_sc_mesh,
    compiler_params=_sc_params,
    scratch_types=[
        pltpu.VMEM((NE_PT,), jnp.int32),
        pltpu.VMEM((NE_PT,), jnp.int32),
        [pltpu.VMEM((GEDGE, DH), jnp.bfloat16)] * NBUF,
        pltpu.VMEM_SHARED((N_PAD, DH), jnp.bfloat16),
        pltpu.VMEM_SHARED((N_PAD, DH), jnp.bfloat16),
        [pltpu.SemaphoreType.DMA] * NBUF,
    ],
)
def _segsum_kernel(u_hbm, src_hbm, dst_hbm, out_hbm,
                   src_v, dst_v, rows_bufs, agg_sh, u_sh, sems):
    c = lax.axis_index("c")
    s = lax.axis_index("s")

    stage_v = rows_bufs[0]
    _spans = ((0, GEDGE), (GEDGE, ROWS_PT - GEDGE))

    def zrow(r, _):
        for k in range(DH // 32):
            stage_v[r, pl.ds(k * 32, 32)] = jnp.zeros((32,), jnp.bfloat16)
        return _

    lax.fori_loop(0, GEDGE, zrow, 0)
    for off, ln in _spans:
        pltpu.sync_copy(stage_v.at[pl.ds(0, ln)],
                        agg_sh.at[pl.ds(s * ROWS_PT + off, ln)])
    sl = pl.ds(s * ROWS_PT, ROWS_PT)
    pltpu.sync_copy(u_hbm.at[c, sl], u_sh.at[sl])
    plsc.subcore_barrier()

    pltpu.sync_copy(src_hbm.at[s], src_v)
    pltpu.sync_copy(dst_hbm.at[s], dst_v)

    def _span(j):
        return pl.ds(j * GEDGE, GEDGE)

    def _table(b):
        return u_sh

    for b in range(NBUF):
        pltpu.async_copy(
            _table(b).at[src_v.at[_span(b)]], rows_bufs[b], sems[b])

    def body(i, _):
        for b in range(NBUF):
            j = i * NBUF + b
            pltpu.make_async_copy(
                _table(b).at[src_v.at[_span(j)]], rows_bufs[b],
                sems[b]).wait()
            pltpu.sync_copy(
                rows_bufs[b], agg_sh.at[dst_v.at[_span(j)]], add=True)

            @pl.when(j + NBUF < N_GRP)
            def _issue():
                pltpu.async_copy(
                    _table(b).at[src_v.at[_span(j + NBUF)]],
                    rows_bufs[b], sems[b])
        return _

    lax.fori_loop(0, N_GRP // NBUF, body, 0)
    plsc.subcore_barrier()
    for off, ln in _spans:
        pltpu.sync_copy(agg_sh.at[pl.ds(s * ROWS_PT + off, ln)],
                        stage_v.at[pl.ds(0, ln)])
        pltpu.sync_copy(stage_v.at[pl.ds(0, ln)],
                        out_hbm.at[c, pl.ds(s * ROWS_PT + off, ln)])


def _norm_col(deg16):
    d = deg16[:N_NODES, 0:1]
    return jnp.where(d > 0, lax.rsqrt(jnp.maximum(d, 1e-12)), 0.0)


def _split_u(u_ref, un):
    ub = un.astype(jnp.bfloat16)
    u_ref[0, :N_NODES, :] = ub[:, :DH]
    u_ref[1, :N_NODES, :] = ub[:, DH:]
    zpad = jnp.zeros((N_PAD - N_NODES, DH), jnp.bfloat16)
    u_ref[0, N_NODES:, :] = zpad
    u_ref[1, N_NODES:, :] = zpad


def _scale_body(deg_ref, x_ref, u_ref):
    _split_u(u_ref, x_ref[...] * _norm_col(deg_ref[0]))


_scale_call = pl.pallas_call(
    _scale_body,
    out_shape=jax.ShapeDtypeStruct((NC, N_PAD, DH), jnp.bfloat16),
)


def _post_body(with_u, aggp_ref, h_ref, deg_ref, wg_ref, bg_ref, wl_ref,
               bl_ref, gm_ref, bt_ref, *outs):
    agg = jnp.concatenate(
        [aggp_ref[0][:N_NODES], aggp_ref[1][:N_NODES]],
        axis=1).astype(jnp.float32)
    h = h_ref[...]
    norm_in = _norm_col(deg_ref[1])
    gcn = jnp.dot(agg, wg_ref[...], preferred_element_type=jnp.float32)
    gcn = gcn * norm_in + bg_ref[...]
    lin = jnp.dot(h, wl_ref[...], preferred_element_type=jnp.float32) + bl_ref[...]
    out = gcn + h + lin
    mean = jnp.mean(out, axis=0, keepdims=True)
    cen = out - mean
    var = jnp.mean(cen * cen, axis=0, keepdims=True)
    hn = cen * lax.rsqrt(var + BN_EPS) * gm_ref[...] + bt_ref[...]
    hn = jnp.maximum(hn, 0.0)
    outs[0][...] = hn
    if with_u:
        _split_u(outs[1], hn * _norm_col(deg_ref[0]))


_post_first = pl.pallas_call(
    functools.partial(_post_body, True),
    out_shape=[jax.ShapeDtypeStruct((N_NODES, D), jnp.float32),
               jax.ShapeDtypeStruct((NC, N_PAD, DH), jnp.bfloat16)],
)
_post_last = pl.pallas_call(
    functools.partial(_post_body, False),
    out_shape=[jax.ShapeDtypeStruct((N_NODES, D), jnp.float32)],
)


def kernel(x, edge_index, W_gcn, b_gcn, W_lin, b_lin, gamma, beta):
    edge_f = edge_index.reshape(2, NS, NE_PT)
    src_r = edge_f[0]
    dst_r = edge_f[1]
    deg16 = _degree_kernel(edge_f)
    u = _scale_call(deg16, x)
    h = x
    for l in range(2):
        aggp = _segsum_kernel(u, src_r, dst_r)
        args = (aggp, h, deg16, W_gcn[l], b_gcn[l].reshape(1, D),
                W_lin[l], b_lin[l].reshape(1, D), gamma[l].reshape(1, D),
                beta[l].reshape(1, D))
        if l == 0:
            h, u = _post_first(*args)
        else:
            (h,) = _post_last(*args)
    return h
